# Initial kernel scaffold; baseline (speedup 1.0000x reference)
import jax
import jax.numpy as jnp
from jax.experimental import pallas as pl


def kernel(z_old, W, b, neighbour_list):
    raise NotImplementedError("write your pallas kernel here")


# speedup vs baseline: 4.7180x; 4.7180x over previous
import functools

import jax
import jax.numpy as jnp
from jax import lax
from jax.experimental import pallas as pl
from jax.experimental.pallas import tpu as pltpu
from jax.experimental.pallas import tpu_sc as plsc

N = 100000
D = 128
DD = 16
NW = 32
NPAD = 100352
CPW = NPAD // NW
SUB = 112
NSUB = CPW // SUB
BLK = 1024
GRID1 = NPAD // BLK
BLK3 = 1000


def _proj_body(z_ref, w_ref, b_ref, p0, p1, p2, p3):
    acc = jnp.dot(z_ref[...], w_ref[...], preferred_element_type=jnp.float32)
    p0[...] = acc[:, 0:16] + b_ref[...]
    p1[...] = acc[:, 16:32]
    p2[...] = acc[:, 32:48]
    p3[...] = acc[:, 48:64]


def _project(z, wstack, b2):
    out = jax.ShapeDtypeStruct((NPAD, DD), jnp.float32)
    return pl.pallas_call(
        _proj_body,
        grid=(GRID1,),
        in_specs=[
            pl.BlockSpec((BLK, D), lambda g: (g, 0)),
            pl.BlockSpec((D, 4 * DD), lambda g: (0, 0)),
            pl.BlockSpec((1, DD), lambda g: (0, 0)),
        ],
        out_specs=[pl.BlockSpec((BLK, DD), lambda g: (g, 0))] * 4,
        out_shape=[out] * 4,
    )(z, wstack, b2)


def _update_body(f_ref, wsm_ref, a0, a1, a2, a3, o0, o1, o2, o3):
    d = jnp.dot(f_ref[...], wsm_ref[...], preferred_element_type=jnp.float32)
    o0[...] = a0[...] + d[:, 0:16]
    o1[...] = a1[...] + d[:, 16:32]
    o2[...] = a2[...] + d[:, 32:48]
    o3[...] = a3[...] + d[:, 48:64]


def _update(f1, wsm, p0, p1, p2, p3):
    out = jax.ShapeDtypeStruct((NPAD, DD), jnp.float32)
    pb = pl.BlockSpec((BLK, DD), lambda g: (g, 0))
    return pl.pallas_call(
        _update_body,
        grid=(GRID1,),
        in_specs=[pb, pl.BlockSpec((DD, 4 * DD), lambda g: (0, 0)),
                  pb, pb, pb, pb],
        out_specs=[pb] * 4,
        out_shape=[out] * 4,
    )(f1, wsm, p0, p1, p2, p3)


def _final_body(z_ref, f1_ref, f2_ref, out_ref):
    s = f1_ref[...] + f2_ref[...]
    zz = z_ref[...]
    out_ref[...] = jnp.concatenate([zz[:, :DD] + s, zz[:, DD:]], axis=1)


def _final(z, f1, f2):
    fb = pl.BlockSpec((BLK3, DD), lambda g: (g, 0))
    zb = pl.BlockSpec((BLK3, D), lambda g: (g, 0))
    return pl.pallas_call(
        _final_body,
        grid=(N // BLK3,),
        in_specs=[zb, fb, fb],
        out_specs=zb,
        out_shape=jax.ShapeDtypeStruct((N, D), jnp.float32),
    )(z, f1, f2)


def _gather_sum(p0, p1, p2, p3, nl3):
    mesh = plsc.VectorSubcoreMesh(core_axis_name="c", subcore_axis_name="s")

    @functools.partial(
        pl.kernel, mesh=mesh,
        compiler_params=pltpu.CompilerParams(use_tc_tiling_on_sc=False),
        out_type=jax.ShapeDtypeStruct((NPAD, DD), jnp.float32),
        scratch_types=[
            pltpu.VMEM((3, NSUB, SUB), jnp.int32),
            pltpu.VMEM((CPW, DD), jnp.float32),
            pltpu.VMEM((SUB, DD), jnp.float32),
            pltpu.VMEM((SUB, DD), jnp.float32),
            pltpu.VMEM((SUB, DD), jnp.float32),
            pltpu.VMEM((SUB, DD), jnp.float32),
            pltpu.SemaphoreType.DMA,
        ],
    )
    def body(p0_h, p1_h, p2_h, p3_h, nl_h, f_h, idx_v, p0_v, r1, r2, r3, fb,
             sem):
        wid = lax.axis_index("s") * 2 + lax.axis_index("c")
        base_w = pl.multiple_of(wid * CPW, CPW)
        pltpu.sync_copy(nl_h.at[wid], idx_v)
        pltpu.sync_copy(p0_h.at[pl.ds(base_w, CPW)], p0_v)

        def sub(s, carry):
            g1 = pltpu.async_copy(p1_h.at[idx_v.at[0, s]], r1, sem)
            g2 = pltpu.async_copy(p2_h.at[idx_v.at[1, s]], r2, sem)
            g3 = pltpu.async_copy(p3_h.at[idx_v.at[2, s]], r3, sem)
            g1.wait()
            g2.wait()
            g3.wait()
            rowbase = s * SUB

            def row(i, c):
                fb[i] = (p0_v[rowbase + i] + r1[i]) + (r2[i] + r3[i])
                return c

            lax.fori_loop(0, SUB, row, 0)
            off = pl.multiple_of(base_w + rowbase, SUB)
            pltpu.sync_copy(fb, f_h.at[pl.ds(off, SUB)])
            return carry

        lax.fori_loop(0, NSUB, sub, 0)

    return body(p0, p1, p2, p3, nl3)


def kernel(z_old, W, b, neighbour_list):
    wstack = jnp.concatenate([W[0::4], W[1::4], W[2::4], W[3::4]], axis=1)
    wsm = wstack[:DD, :]
    b2 = b.reshape(1, DD)
    nl_pad = jnp.zeros((3, NPAD), jnp.int32).at[:, :N].set(neighbour_list.T)
    nl3 = jnp.transpose(nl_pad.reshape(3, NW, NSUB, SUB), (1, 0, 2, 3))

    p0, p1, p2, p3 = _project(z_old, wstack, b2)
    f1 = _gather_sum(p0, p1, p2, p3, nl3)
    q0, q1, q2, q3 = _update(f1, wsm, p0, p1, p2, p3)
    f2 = _gather_sum(q0, q1, q2, q3, nl3)
    return _final(z_old, f1, f2)


# speedup vs baseline: 7.3745x; 1.5630x over previous
import functools

import jax
import jax.numpy as jnp
from jax import lax
from jax.experimental import pallas as pl
from jax.experimental.pallas import tpu as pltpu
from jax.experimental.pallas import tpu_sc as plsc

N = 100000
D = 128
DD = 16
NW = 32
NPAD = 100352
CPW = NPAD // NW
SUB = 112
NSUB = CPW // SUB
BLK = 2048
BLK3 = 5000

_SC_PARAMS = pltpu.CompilerParams(use_tc_tiling_on_sc=False)


def _proj_body(z_ref, w_ref, b_ref, p0, p1, p2, p3):
    acc = jnp.dot(z_ref[...], w_ref[...], preferred_element_type=jnp.float32)
    p0[...] = acc[:, 0:16] + b_ref[...]
    p1[...] = acc[:, 16:32]
    p2[...] = acc[:, 32:48]
    p3[...] = acc[:, 48:64]


def _project(z, wstack, b2):
    out = jax.ShapeDtypeStruct((NPAD, DD), jnp.float32)
    return pl.pallas_call(
        _proj_body,
        grid=(NPAD // BLK,),
        in_specs=[
            pl.BlockSpec((BLK, D), lambda g: (g, 0)),
            pl.BlockSpec((D, 4 * DD), lambda g: (0, 0)),
            pl.BlockSpec((1, DD), lambda g: (0, 0)),
        ],
        out_specs=[pl.BlockSpec((BLK, DD), lambda g: (g, 0))] * 4,
        out_shape=[out] * 4,
    )(z, wstack, b2)


def _final_body(z_ref, f1_ref, h1_ref, h2_ref, h3_ref, wv_ref, out_ref):
    f1 = f1_ref[...]
    cat = jnp.concatenate([f1, h1_ref[...], h2_ref[...], h3_ref[...]], axis=1)
    s = 2.0 * f1 + jnp.dot(cat, wv_ref[...],
                           preferred_element_type=jnp.float32)
    zz = z_ref[...]
    out_ref[...] = jnp.concatenate([zz[:, :DD] + s, zz[:, DD:]], axis=1)


def _final(z, f1, h1, h2, h3, wv):
    fb = pl.BlockSpec((BLK3, DD), lambda g: (g, 0))
    zb = pl.BlockSpec((BLK3, D), lambda g: (g, 0))
    return pl.pallas_call(
        _final_body,
        grid=(N // BLK3,),
        in_specs=[zb, fb, fb, fb, fb,
                  pl.BlockSpec((4 * DD, DD), lambda g: (0, 0))],
        out_specs=zb,
        out_shape=jax.ShapeDtypeStruct((N, D), jnp.float32),
    )(z, f1, h1, h2, h3, wv)


def _gather_sum(p0, p1, p2, p3, nl3):
    mesh = plsc.VectorSubcoreMesh(core_axis_name="c", subcore_axis_name="s")

    @functools.partial(
        pl.kernel, mesh=mesh, compiler_params=_SC_PARAMS,
        out_type=jax.ShapeDtypeStruct((NPAD, DD), jnp.float32),
        scratch_types=[
            pltpu.VMEM((3, NSUB, SUB), jnp.int32),
            pltpu.VMEM((CPW, DD), jnp.float32),
            pltpu.VMEM((SUB, DD), jnp.float32),
            pltpu.VMEM((SUB, DD), jnp.float32),
            pltpu.VMEM((SUB, DD), jnp.float32),
            pltpu.VMEM((SUB, DD), jnp.float32),
            pltpu.SemaphoreType.DMA,
        ],
    )
    def body(p0_h, p1_h, p2_h, p3_h, nl_h, f_h, idx_v, p0_v, r1, r2, r3, fb,
             sem):
        wid = lax.axis_index("s") * 2 + lax.axis_index("c")
        base_w = pl.multiple_of(wid * CPW, CPW)
        pltpu.sync_copy(nl_h.at[wid], idx_v)
        pltpu.sync_copy(p0_h.at[pl.ds(base_w, CPW)], p0_v)

        def sub(s, carry):
            g1 = pltpu.async_copy(p1_h.at[idx_v.at[0, s]], r1, sem)
            g2 = pltpu.async_copy(p2_h.at[idx_v.at[1, s]], r2, sem)
            g3 = pltpu.async_copy(p3_h.at[idx_v.at[2, s]], r3, sem)
            g1.wait()
            g2.wait()
            g3.wait()
            rowbase = s * SUB

            def row(i, c):
                fb[i] = (p0_v[rowbase + i] + r1[i]) + (r2[i] + r3[i])
                return c

            lax.fori_loop(0, SUB, row, 0)
            off = pl.multiple_of(base_w + rowbase, SUB)
            pltpu.sync_copy(fb, f_h.at[pl.ds(off, SUB)])
            return carry

        lax.fori_loop(0, NSUB, sub, 0)

    return body(p0, p1, p2, p3, nl3)


def _gather3(f1, nl3):
    mesh = plsc.VectorSubcoreMesh(core_axis_name="c", subcore_axis_name="s")
    ht = jax.ShapeDtypeStruct((NPAD, DD), jnp.float32)

    @functools.partial(
        pl.kernel, mesh=mesh, compiler_params=_SC_PARAMS,
        out_type=[ht, ht, ht],
        scratch_types=[
            pltpu.VMEM((3, NSUB, SUB), jnp.int32),
            pltpu.VMEM((SUB, DD), jnp.float32),
            pltpu.VMEM((SUB, DD), jnp.float32),
            pltpu.VMEM((SUB, DD), jnp.float32),
            pltpu.SemaphoreType.DMA,
        ],
    )
    def body(f1_h, nl_h, h1_h, h2_h, h3_h, idx_v, r1, r2, r3, sem):
        wid = lax.axis_index("s") * 2 + lax.axis_index("c")
        base_w = pl.multiple_of(wid * CPW, CPW)
        pltpu.sync_copy(nl_h.at[wid], idx_v)

        def sub(s, carry):
            g1 = pltpu.async_copy(f1_h.at[idx_v.at[0, s]], r1, sem)
            g2 = pltpu.async_copy(f1_h.at[idx_v.at[1, s]], r2, sem)
            g3 = pltpu.async_copy(f1_h.at[idx_v.at[2, s]], r3, sem)
            off = pl.multiple_of(base_w + s * SUB, SUB)
            g1.wait()
            pltpu.sync_copy(r1, h1_h.at[pl.ds(off, SUB)])
            g2.wait()
            pltpu.sync_copy(r2, h2_h.at[pl.ds(off, SUB)])
            g3.wait()
            pltpu.sync_copy(r3, h3_h.at[pl.ds(off, SUB)])
            return carry

        lax.fori_loop(0, NSUB, sub, 0)

    return body(f1, nl3)


def kernel(z_old, W, b, neighbour_list):
    w0, w1, w2, w3 = W[0::4], W[1::4], W[2::4], W[3::4]
    wstack = jnp.concatenate([w0, w1, w2, w3], axis=1)
    wv = jnp.concatenate([w0[:DD], w1[:DD], w2[:DD], w3[:DD]], axis=0)
    b2 = b.reshape(1, DD)
    nl_pad = jnp.zeros((3, NPAD), jnp.int32).at[:, :N].set(neighbour_list.T)
    nl3 = jnp.transpose(nl_pad.reshape(3, NW, NSUB, SUB), (1, 0, 2, 3))

    p0, p1, p2, p3 = _project(z_old, wstack, b2)
    f1 = _gather_sum(p0, p1, p2, p3, nl3)
    h1, h2, h3 = _gather3(f1, nl3)
    return _final(z_old, f1, h1, h2, h3, wv)


# speedup vs baseline: 11.6850x; 1.5845x over previous
import functools

import jax
import jax.numpy as jnp
from jax import lax
from jax.experimental import pallas as pl
from jax.experimental.pallas import tpu as pltpu
from jax.experimental.pallas import tpu_sc as plsc

N = 100000
D = 128
DD = 16
NW = 32
NPAD = 100352
CPW = NPAD // NW
SUB = 112
NSUB = CPW // SUB
NROW = NPAD // 8
NZROW = N // 8
ZW = 8 * D
BLKR = 448

_SC_PARAMS = pltpu.CompilerParams(use_tc_tiling_on_sc=False)


def _proj_body(zf_ref, w_ref, b_ref, p0, p1, p2, p3):
    zf = zf_ref[...]
    w = w_ref[...]
    accs = [
        jnp.dot(zf[:, g * D:(g + 1) * D], w,
                preferred_element_type=jnp.float32)
        for g in range(8)
    ]
    p0[...] = jnp.concatenate([a[:, 0:16] for a in accs], axis=1) + b_ref[...]
    p1[...] = jnp.concatenate([a[:, 16:32] for a in accs], axis=1)
    p2[...] = jnp.concatenate([a[:, 32:48] for a in accs], axis=1)
    p3[...] = jnp.concatenate([a[:, 48:64] for a in accs], axis=1)


def _project(zf, wstack, b128):
    out = jax.ShapeDtypeStruct((NROW, D), jnp.float32)
    return pl.pallas_call(
        _proj_body,
        grid=(NROW // BLKR,),
        in_specs=[
            pl.BlockSpec((BLKR, ZW), lambda g: (g, 0)),
            pl.BlockSpec((D, 4 * DD), lambda g: (0, 0)),
            pl.BlockSpec((1, D), lambda g: (0, 0)),
        ],
        out_specs=[pl.BlockSpec((BLKR, D), lambda g: (g, 0))] * 4,
        out_shape=[out] * 4,
    )(zf, wstack, b128)


def _final_body(zf_ref, f1_ref, h1_ref, h2_ref, h3_ref, k0_ref, k1_ref,
                k2_ref, k3_ref, out_ref):
    f1 = f1_ref[...]
    s = (2.0 * f1
         + jnp.dot(f1, k0_ref[...], preferred_element_type=jnp.float32)
         + jnp.dot(h1_ref[...], k1_ref[...], preferred_element_type=jnp.float32)
         + jnp.dot(h2_ref[...], k2_ref[...], preferred_element_type=jnp.float32)
         + jnp.dot(h3_ref[...], k3_ref[...], preferred_element_type=jnp.float32))
    zf = zf_ref[...]
    pieces = []
    for g in range(8):
        pieces.append(zf[:, g * D:g * D + DD] + s[:, g * DD:(g + 1) * DD])
        pieces.append(zf[:, g * D + DD:(g + 1) * D])
    out_ref[...] = jnp.concatenate(pieces, axis=1)


def _final(zf, f1, h1, h2, h3, kmats):
    fb = pl.BlockSpec((BLKR, D), lambda g: (g, 0))
    zb = pl.BlockSpec((BLKR, ZW), lambda g: (g, 0))
    kb = pl.BlockSpec((D, D), lambda g: (0, 0))
    return pl.pallas_call(
        _final_body,
        grid=(NROW // BLKR,),
        in_specs=[zb, fb, fb, fb, fb, kb, kb, kb, kb],
        out_specs=zb,
        out_shape=jax.ShapeDtypeStruct((NZROW, ZW), jnp.float32),
    )(zf, f1, h1, h2, h3, *kmats)


def _gather_sum(p0, p1, p2, p3, nl3):
    mesh = plsc.VectorSubcoreMesh(core_axis_name="c", subcore_axis_name="s")

    @functools.partial(
        pl.kernel, mesh=mesh, compiler_params=_SC_PARAMS,
        out_type=jax.ShapeDtypeStruct((NPAD, DD), jnp.float32),
        scratch_types=[
            pltpu.VMEM((3, NSUB, SUB), jnp.int32),
            pltpu.VMEM((CPW, DD), jnp.float32),
            pltpu.VMEM((SUB, DD), jnp.float32),
            pltpu.VMEM((SUB, DD), jnp.float32),
            pltpu.VMEM((SUB, DD), jnp.float32),
            pltpu.VMEM((SUB, DD), jnp.float32),
            pltpu.SemaphoreType.DMA,
        ],
    )
    def body(p0_h, p1_h, p2_h, p3_h, nl_h, f_h, idx_v, p0_v, r1, r2, r3, fb,
             sem):
        wid = lax.axis_index("s") * 2 + lax.axis_index("c")
        base_w = pl.multiple_of(wid * CPW, CPW)
        pltpu.sync_copy(nl_h.at[wid], idx_v)
        pltpu.sync_copy(p0_h.at[pl.ds(base_w, CPW)], p0_v)

        def sub(s, carry):
            g1 = pltpu.async_copy(p1_h.at[idx_v.at[0, s]], r1, sem)
            g2 = pltpu.async_copy(p2_h.at[idx_v.at[1, s]], r2, sem)
            g3 = pltpu.async_copy(p3_h.at[idx_v.at[2, s]], r3, sem)
            g1.wait()
            g2.wait()
            g3.wait()
            rowbase = s * SUB

            def row(i, c):
                fb[i] = (p0_v[rowbase + i] + r1[i]) + (r2[i] + r3[i])
                return c

            lax.fori_loop(0, SUB, row, 0)
            off = pl.multiple_of(base_w + rowbase, SUB)
            pltpu.sync_copy(fb, f_h.at[pl.ds(off, SUB)])
            return carry

        lax.fori_loop(0, NSUB, sub, 0)

    return body(p0, p1, p2, p3, nl3)


def _gather3(f1, nl3):
    mesh = plsc.VectorSubcoreMesh(core_axis_name="c", subcore_axis_name="s")
    ht = jax.ShapeDtypeStruct((NPAD, DD), jnp.float32)

    @functools.partial(
        pl.kernel, mesh=mesh, compiler_params=_SC_PARAMS,
        out_type=[ht, ht, ht],
        scratch_types=[
            pltpu.VMEM((3, NSUB, SUB), jnp.int32),
            pltpu.VMEM((SUB, DD), jnp.float32),
            pltpu.VMEM((SUB, DD), jnp.float32),
            pltpu.VMEM((SUB, DD), jnp.float32),
            pltpu.SemaphoreType.DMA,
        ],
    )
    def body(f1_h, nl_h, h1_h, h2_h, h3_h, idx_v, r1, r2, r3, sem):
        wid = lax.axis_index("s") * 2 + lax.axis_index("c")
        base_w = pl.multiple_of(wid * CPW, CPW)
        pltpu.sync_copy(nl_h.at[wid], idx_v)

        def sub(s, carry):
            g1 = pltpu.async_copy(f1_h.at[idx_v.at[0, s]], r1, sem)
            g2 = pltpu.async_copy(f1_h.at[idx_v.at[1, s]], r2, sem)
            g3 = pltpu.async_copy(f1_h.at[idx_v.at[2, s]], r3, sem)
            off = pl.multiple_of(base_w + s * SUB, SUB)
            g1.wait()
            pltpu.sync_copy(r1, h1_h.at[pl.ds(off, SUB)])
            g2.wait()
            pltpu.sync_copy(r2, h2_h.at[pl.ds(off, SUB)])
            g3.wait()
            pltpu.sync_copy(r3, h3_h.at[pl.ds(off, SUB)])
            return carry

        lax.fori_loop(0, NSUB, sub, 0)

    return body(f1, nl3)


def kernel(z_old, W, b, neighbour_list):
    w0, w1, w2, w3 = W[0::4], W[1::4], W[2::4], W[3::4]
    wstack = jnp.concatenate([w0, w1, w2, w3], axis=1)
    b128 = jnp.tile(b, 8).reshape(1, D)
    eye8 = jnp.eye(8, dtype=jnp.float32)
    kmats = [jnp.kron(eye8, wk[:DD]) for wk in (w0, w1, w2, w3)]
    nl_pad = jnp.zeros((3, NPAD), jnp.int32).at[:, :N].set(neighbour_list.T)
    nl3 = jnp.transpose(nl_pad.reshape(3, NW, NSUB, SUB), (1, 0, 2, 3))

    flat = lambda a: jnp.reshape(a, (NPAD, DD))
    wide = lambda a: jnp.reshape(a, (NROW, D))
    zf = jnp.reshape(z_old, (NZROW, ZW))
    p0, p1, p2, p3 = _project(zf, wstack, b128)
    f1 = _gather_sum(flat(p0), flat(p1), flat(p2), flat(p3), nl3)
    h1, h2, h3 = _gather3(f1, nl3)
    out = _final(zf, wide(f1), wide(h1), wide(h2), wide(h3), kmats)
    return jnp.reshape(out, (N, D))
